# baseline (speedup 1.0000x reference)
import functools

import jax
import jax.numpy as jnp
from jax import lax
from jax.experimental import pallas as pl
from jax.experimental.pallas import tpu as pltpu
from jax.experimental.pallas import tpu_sc as plsc

_HS = (56, 28, 14, 7)
_CS = (64, 128, 256, 512)
_SCALES = (4.0, 8.0, 16.0, 32.0)
_COL0 = (3, 67, 195, 451)
_NCOLS = 3 + sum(_CS)
_GW = (128, 128, 256, 512)
_B = 16


_GDN = lax.GatherDimensionNumbers(
    offset_dims=(), collapsed_slice_dims=(0,), start_index_map=(0,))


def _permute(vec, idx):
    return lax.gather(vec, idx[:, None], _GDN, (1,),
                      mode=lax.GatherScatterMode.PROMISE_IN_BOUNDS)


def _bcast(vec, i):
    return _permute(vec, jnp.zeros((16,), jnp.int32) + i)


def _make_sc_call(n):
    nw = 32
    rows_per_w = -(-n // (nw * _B)) * _B
    n_pad = nw * rows_per_w
    mesh = plsc.VectorSubcoreMesh(core_axis_name="c", subcore_axis_name="s")

    @functools.partial(
        pl.kernel,
        mesh=mesh,
        out_type=jax.ShapeDtypeStruct((n * _NCOLS,), jnp.float32),
        scratch_types=[
            pltpu.VMEM((rows_per_w,), jnp.float32),
            pltpu.VMEM((rows_per_w,), jnp.float32),
            pltpu.VMEM((rows_per_w,), jnp.float32),
            pltpu.VMEM((4 * _B, _GW[0]), jnp.float32),
            pltpu.VMEM((4 * _B, _GW[1]), jnp.float32),
            pltpu.VMEM((4 * _B, _GW[2]), jnp.float32),
            pltpu.VMEM((4 * _B, _GW[3]), jnp.float32),
            pltpu.VMEM((4 * _B,), jnp.int32),
            pltpu.VMEM((4 * _B,), jnp.int32),
            pltpu.VMEM((4 * _B,), jnp.int32),
            pltpu.VMEM((4 * _B,), jnp.int32),
            pltpu.VMEM((_B * _NCOLS,), jnp.float32),
            pltpu.SemaphoreType.DMA,
        ],
    )
    def sc_call(x_hbm, y_hbm, z_hbm, t0, t1, t2, t3, out_hbm,
                x_v, y_v, z_v, q0, q1, q2, q3,
                i0, i1, i2, i3, o_v, sem):
        tabs = (t0, t1, t2, t3)
        qs = (q0, q1, q2, q3)
        idxs = (i0, i1, i2, i3)
        wid = lax.axis_index("s") * 2 + lax.axis_index("c")
        base = wid * rows_per_w
        nb = jnp.minimum(rows_per_w, n - base) // _B
        pltpu.sync_copy(x_hbm.at[pl.ds(base, rows_per_w)], x_v)
        pltpu.sync_copy(y_hbm.at[pl.ds(base, rows_per_w)], y_v)
        pltpu.sync_copy(z_hbm.at[pl.ds(base, rows_per_w)], z_v)

        lane = lax.iota(jnp.int32, _B)
        lane_m3 = jnp.maximum(lane - 3, 0)

        def batch_body(b, carry):
            off = b * _B
            xb = x_v[pl.ds(off, _B)]
            yb = y_v[pl.ds(off, _B)]
            zb = z_v[pl.ds(off, _B)]
            h = 248.0 * (yb / zb) + 112.0
            w = 248.0 * (xb / (-zb)) + 112.0
            h = jnp.clip(h, 0.0, 223.0)
            w = jnp.clip(w, 0.0, 223.0)
            weights = []
            for l in range(4):
                hh = _HS[l]
                x = h / _SCALES[l]
                y = w / _SCALES[l]
                xi1 = x.astype(jnp.int32)
                x1f = xi1.astype(jnp.float32)
                xfr = x > x1f
                xi2 = xi1 + jnp.where(xfr, 1, 0)
                x2f = x1f + jnp.where(xfr, 1.0, 0.0)
                yi1 = y.astype(jnp.int32)
                y1f = yi1.astype(jnp.float32)
                yfr = y > y1f
                yi2 = yi1 + jnp.where(yfr, 1, 0)
                y2f = y1f + jnp.where(yfr, 1.0, 0.0)
                xi2c = jnp.minimum(xi2, hh - 1)
                yi2c = jnp.minimum(yi2, hh - 1)
                weights.append(((x2f - x) * (y2f - y), (x - x1f) * (y2f - y),
                                (x2f - x) * (y - y1f), (x - x1f) * (y - y1f)))
                idxs[l][pl.ds(0, _B)] = xi1 * hh + yi1
                idxs[l][pl.ds(_B, _B)] = xi2c * hh + yi1
                idxs[l][pl.ds(2 * _B, _B)] = xi1 * hh + yi2c
                idxs[l][pl.ds(3 * _B, _B)] = xi2c * hh + yi2c
            copies = [pltpu.async_copy(tabs[l].at[idxs[l]], qs[l], sem)
                      for l in range(4)]
            for c in copies:
                c.wait()

            def point_body(i, carry2):
                rbase = i * _NCOLS
                for l in range(4):
                    q = qs[l]
                    w11 = _bcast(weights[l][0], i)
                    w21 = _bcast(weights[l][1], i)
                    w12 = _bcast(weights[l][2], i)
                    w22 = _bcast(weights[l][3], i)
                    for k in range(_CS[l] // 16):
                        sl = pl.ds(k * 16, 16)
                        v = (q[i, sl] * w11 + q[i + _B, sl] * w21
                             + q[i + 2 * _B, sl] * w12 + q[i + 3 * _B, sl] * w22)
                        if l == 0 and k == 0:
                            head = jnp.where(
                                lane == 0, _bcast(xb, i),
                                jnp.where(lane == 1, _bcast(yb, i),
                                          jnp.where(lane == 2, _bcast(zb, i),
                                                    _permute(v, lane_m3))))
                            o_v[pl.ds(rbase, 16)] = head
                        o_v[pl.ds(rbase + _COL0[l] + k * 16, 16)] = v
                return carry2

            lax.fori_loop(0, _B, point_body, 0)
            pltpu.sync_copy(o_v, out_hbm.at[pl.ds((base + off) * _NCOLS, _B * _NCOLS)])
            return carry

        lax.fori_loop(0, nb, batch_body, 0)

    return sc_call


@jax.jit
def kernel(inputs, img_feat_0, img_feat_1, img_feat_2, img_feat_3):
    n = inputs.shape[0]
    nw = 32
    rows_per_w = -(-n // (nw * _B)) * _B
    n_pad = nw * rows_per_w
    xyz = jnp.pad(inputs.T, ((0, 0), (0, n_pad - n)), constant_values=-1.0)
    tables = [
        t.reshape(hh * hh, cc)
        for t, hh, cc in zip((img_feat_0, img_feat_1, img_feat_2, img_feat_3), _HS, _CS)
    ]
    tables = [
        jnp.pad(t, ((0, 0), (0, gw - t.shape[1]))) if gw != t.shape[1] else t
        for t, gw in zip(tables, _GW)
    ]
    out_flat = _make_sc_call(n)(xyz[0], xyz[1], xyz[2], *tables)
    return out_flat.reshape(n, _NCOLS)


# speedup vs baseline: 1.2946x; 1.2946x over previous
import functools

import jax
import jax.numpy as jnp
from jax import lax
from jax.experimental import pallas as pl
from jax.experimental.pallas import tpu as pltpu
from jax.experimental.pallas import tpu_sc as plsc

_HS = (56, 28, 14, 7)
_CS = (64, 128, 256, 512)
_SCALES = (4.0, 8.0, 16.0, 32.0)
_COL0 = (3, 67, 195, 451)
_NCOLS = 3 + sum(_CS)
_QDT = (jnp.float32, jnp.float32, jnp.int32, jnp.int32)
_GW = (128, 128, 128, 256)
_B = 16

_GDN = lax.GatherDimensionNumbers(
    offset_dims=(), collapsed_slice_dims=(0,), start_index_map=(0,))


def _permute(vec, idx):
    return lax.gather(vec, idx[:, None], _GDN, (1,),
                      mode=lax.GatherScatterMode.PROMISE_IN_BOUNDS)


def _bcast(vec, i):
    return _permute(vec, jnp.zeros((16,), jnp.int32) + i)


def _halves(word):
    a = jax.lax.bitcast_convert_type(word << 16, jnp.float32)
    b = jax.lax.bitcast_convert_type(word & jnp.int32(-65536), jnp.float32)
    return a, b


def _make_sc_call(n):
    nw = 32
    rows_per_w = -(-n // (nw * _B)) * _B
    nb_max = rows_per_w // _B
    mesh = plsc.VectorSubcoreMesh(core_axis_name="c", subcore_axis_name="s")

    qtypes = [pltpu.VMEM((4 * _B, gw), dt) for gw, dt in zip(_GW, _QDT)]
    itypes = [pltpu.VMEM((4 * _B,), jnp.int32) for _ in range(4)]

    @functools.partial(
        pl.kernel,
        mesh=mesh,
        out_type=jax.ShapeDtypeStruct((n * _NCOLS,), jnp.float32),
        scratch_types=[
            pltpu.VMEM((rows_per_w + _B,), jnp.float32),
            pltpu.VMEM((rows_per_w + _B,), jnp.float32),
            pltpu.VMEM((rows_per_w + _B,), jnp.float32),
        ]
        + qtypes + qtypes + itypes + itypes
        + [
            pltpu.VMEM((_B * _NCOLS,), jnp.float32),
            pltpu.SemaphoreType.DMA,
            pltpu.SemaphoreType.DMA,
        ],
    )
    def sc_call(x_hbm, y_hbm, z_hbm, t0, t1, t2, t3, out_hbm,
                x_v, y_v, z_v,
                qa0, qa1, qa2, qa3, qb0, qb1, qb2, qb3,
                ia0, ia1, ia2, ia3, ib0, ib1, ib2, ib3,
                o_v, sem_a, sem_b):
        tabs = (t0, t1, t2, t3)
        qsets = ((qa0, qa1, qa2, qa3), (qb0, qb1, qb2, qb3))
        isets = ((ia0, ia1, ia2, ia3), (ib0, ib1, ib2, ib3))
        sems = (sem_a, sem_b)
        wid = lax.axis_index("s") * 2 + lax.axis_index("c")
        base = wid * rows_per_w
        nb = jnp.minimum(rows_per_w, n - base) // _B
        pltpu.sync_copy(x_hbm.at[pl.ds(base, rows_per_w)],
                        x_v.at[pl.ds(0, rows_per_w)])
        pltpu.sync_copy(y_hbm.at[pl.ds(base, rows_per_w)],
                        y_v.at[pl.ds(0, rows_per_w)])
        pltpu.sync_copy(z_hbm.at[pl.ds(base, rows_per_w)],
                        z_v.at[pl.ds(0, rows_per_w)])

        lane = lax.iota(jnp.int32, _B)
        lane_m3 = jnp.maximum(lane - 3, 0)

        def fire(b, s):
            off = b * _B
            xb = x_v[pl.ds(off, _B)]
            yb = y_v[pl.ds(off, _B)]
            zb = z_v[pl.ds(off, _B)]
            h = 248.0 * (yb / zb) + 112.0
            w = 248.0 * (xb / (-zb)) + 112.0
            h = jnp.clip(h, 0.0, 223.0)
            w = jnp.clip(w, 0.0, 223.0)
            wts = []
            for l in range(4):
                hh = _HS[l]
                x = h / _SCALES[l]
                y = w / _SCALES[l]
                xi1 = x.astype(jnp.int32)
                x1f = xi1.astype(jnp.float32)
                xfr = x > x1f
                xi2 = xi1 + jnp.where(xfr, 1, 0)
                x2f = x1f + jnp.where(xfr, 1.0, 0.0)
                yi1 = y.astype(jnp.int32)
                y1f = yi1.astype(jnp.float32)
                yfr = y > y1f
                yi2 = yi1 + jnp.where(yfr, 1, 0)
                y2f = y1f + jnp.where(yfr, 1.0, 0.0)
                xi1 = jnp.clip(xi1, 0, hh - 1)
                xi2 = jnp.clip(xi2, 0, hh - 1)
                yi1 = jnp.clip(yi1, 0, hh - 1)
                yi2 = jnp.clip(yi2, 0, hh - 1)
                wts.extend(((x2f - x) * (y2f - y), (x - x1f) * (y2f - y),
                            (x2f - x) * (y - y1f), (x - x1f) * (y - y1f)))
                iref = isets[s][l]
                iref[pl.ds(0, _B)] = xi1 * hh + yi1
                iref[pl.ds(_B, _B)] = xi2 * hh + yi1
                iref[pl.ds(2 * _B, _B)] = xi1 * hh + yi2
                iref[pl.ds(3 * _B, _B)] = xi2 * hh + yi2
            for l in range(4):
                pltpu.async_copy(tabs[l].at[isets[s][l]], qsets[s][l], sems[s])
            return wts + [xb, yb, zb]

        def compute(b, s, state):
            qs = qsets[s]
            xb, yb, zb = state[16], state[17], state[18]

            def point_body(i, carry2):
                rbase = i * _NCOLS
                for l in range(4):
                    q = qs[l]
                    w11 = _bcast(state[4 * l + 0], i)
                    w21 = _bcast(state[4 * l + 1], i)
                    w12 = _bcast(state[4 * l + 2], i)
                    w22 = _bcast(state[4 * l + 3], i)
                    if l < 2:
                        for k in range(_CS[l] // 16):
                            sl = pl.ds(k * 16, 16)
                            v = (q[i, sl] * w11 + q[i + _B, sl] * w21
                                 + q[i + 2 * _B, sl] * w12
                                 + q[i + 3 * _B, sl] * w22)
                            if l == 0 and k == 0:
                                head = jnp.where(
                                    lane == 0, _bcast(xb, i),
                                    jnp.where(lane == 1, _bcast(yb, i),
                                              jnp.where(lane == 2, _bcast(zb, i),
                                                        _permute(v, lane_m3))))
                                o_v[pl.ds(rbase, 16)] = head
                            o_v[pl.ds(rbase + _COL0[l] + k * 16, 16)] = v
                    else:
                        for k in range(_CS[l] // 32):
                            sl = pl.ds(k * 16, 16)
                            a11, b11 = _halves(q[i, sl])
                            a21, b21 = _halves(q[i + _B, sl])
                            a12, b12 = _halves(q[i + 2 * _B, sl])
                            a22, b22 = _halves(q[i + 3 * _B, sl])
                            va = a11 * w11 + a21 * w21 + a12 * w12 + a22 * w22
                            vb = b11 * w11 + b21 * w21 + b12 * w12 + b22 * w22
                            o_v[pl.ds(rbase + _COL0[l] + k * 32, 16)] = va
                            o_v[pl.ds(rbase + _COL0[l] + k * 32 + 16, 16)] = vb
                return carry2

            lax.fori_loop(0, _B, point_body, 0)

            @pl.when(b < nb)
            def _():
                pltpu.sync_copy(
                    o_v, out_hbm.at[pl.ds((base + b * _B) * _NCOLS, _B * _NCOLS)])

        def wait_set(s):
            for l in range(4):
                pltpu.make_async_copy(tabs[l].at[isets[s][l]], qsets[s][l],
                                      sems[s]).wait()

        state0 = fire(0, 0)

        def pair_body(g, carry):
            state = carry
            b0 = 2 * g
            state_n = fire(b0 + 1, 1)
            wait_set(0)
            compute(b0, 0, state)
            state_nn = fire(b0 + 2, 0)
            wait_set(1)
            compute(b0 + 1, 1, state_n)
            return state_nn

        lax.fori_loop(0, nb_max // 2, pair_body, state0)
        wait_set(0)

    return sc_call


def _prep_table(t, hh, cc, gw, dt):
    t = t.reshape(hh * hh, cc)
    if dt == jnp.float32:
        if gw != cc:
            t = jnp.pad(t, ((0, 0), (0, gw - cc)))
        return t
    tu = jax.lax.bitcast_convert_type(
        t.astype(jnp.bfloat16), jnp.uint16).astype(jnp.uint32)
    tu = tu.reshape(hh * hh, cc // 32, 2, 16)
    word = tu[:, :, 0, :] | (tu[:, :, 1, :] << 16)
    return jax.lax.bitcast_convert_type(word, jnp.int32).reshape(
        hh * hh, cc // 2)


@jax.jit
def kernel(inputs, img_feat_0, img_feat_1, img_feat_2, img_feat_3):
    n = inputs.shape[0]
    nw = 32
    rows_per_w = -(-n // (nw * _B)) * _B
    n_pad = nw * rows_per_w
    xyz = jnp.pad(inputs.T, ((0, 0), (0, n_pad - n)), constant_values=-1.0)
    tables = [
        _prep_table(t, hh, cc, gw, dt)
        for t, hh, cc, gw, dt in zip(
            (img_feat_0, img_feat_1, img_feat_2, img_feat_3),
            _HS, _CS, _GW, _QDT)
    ]
    out_flat = _make_sc_call(n)(xyz[0], xyz[1], xyz[2], *tables)
    return out_flat.reshape(n, _NCOLS)


# speedup vs baseline: 1.9041x; 1.4707x over previous
import functools

import jax
import jax.numpy as jnp
from jax import lax
from jax.experimental import pallas as pl
from jax.experimental.pallas import tpu as pltpu
from jax.experimental.pallas import tpu_sc as plsc

_HS = (56, 28, 14, 7)
_CS = (64, 128, 256, 512)
_SCALES = (4.0, 8.0, 16.0, 32.0)
_COL0 = (3, 67, 195, 451)
_NCOLS = 3 + sum(_CS)
_QDT = (jnp.float32, jnp.float32, jnp.int32, jnp.int32)
_GW = (128, 128, 128, 256)
_B = 16

_GDN = lax.GatherDimensionNumbers(
    offset_dims=(), collapsed_slice_dims=(0,), start_index_map=(0,))


def _permute(vec, idx):
    return lax.gather(vec, idx[:, None], _GDN, (1,),
                      mode=lax.GatherScatterMode.PROMISE_IN_BOUNDS)


def _bcast(vec, i):
    return _permute(vec, jnp.zeros((16,), jnp.int32) + i)


def _halves(word):
    a = jax.lax.bitcast_convert_type(word << 16, jnp.float32)
    b = jax.lax.bitcast_convert_type(word & jnp.int32(-65536), jnp.float32)
    return a, b


def _make_sc_call(n):
    nw = 32
    rows_per_w = -(-n // (nw * _B)) * _B
    nb_max = rows_per_w // _B
    mesh = plsc.VectorSubcoreMesh(core_axis_name="c", subcore_axis_name="s")

    qtypes = [pltpu.VMEM((4 * _B, gw), dt) for gw, dt in zip(_GW, _QDT)]
    itypes = [pltpu.VMEM((4 * _B,), jnp.int32) for _ in range(4)]

    @functools.partial(
        pl.kernel,
        mesh=mesh,
        out_type=jax.ShapeDtypeStruct((n, _NCOLS), jnp.float32),
        scratch_types=[
            pltpu.VMEM((rows_per_w + _B,), jnp.float32),
            pltpu.VMEM((rows_per_w + _B,), jnp.float32),
            pltpu.VMEM((rows_per_w + _B,), jnp.float32),
        ]
        + qtypes + qtypes + itypes + itypes
        + [
            pltpu.VMEM((_B, _NCOLS), jnp.float32),
            pltpu.SemaphoreType.DMA,
            pltpu.SemaphoreType.DMA,
        ],
    )
    def sc_call(x_hbm, y_hbm, z_hbm, t0, t1, t2, t3, out_hbm,
                x_v, y_v, z_v,
                qa0, qa1, qa2, qa3, qb0, qb1, qb2, qb3,
                ia0, ia1, ia2, ia3, ib0, ib1, ib2, ib3,
                o_v, sem_a, sem_b):
        tabs = (t0, t1, t2, t3)
        qsets = ((qa0, qa1, qa2, qa3), (qb0, qb1, qb2, qb3))
        isets = ((ia0, ia1, ia2, ia3), (ib0, ib1, ib2, ib3))
        sems = (sem_a, sem_b)
        wid = lax.axis_index("s") * 2 + lax.axis_index("c")
        base = wid * rows_per_w
        nb = jnp.minimum(rows_per_w, n - base) // _B
        pltpu.sync_copy(x_hbm.at[pl.ds(base, rows_per_w)],
                        x_v.at[pl.ds(0, rows_per_w)])
        pltpu.sync_copy(y_hbm.at[pl.ds(base, rows_per_w)],
                        y_v.at[pl.ds(0, rows_per_w)])
        pltpu.sync_copy(z_hbm.at[pl.ds(base, rows_per_w)],
                        z_v.at[pl.ds(0, rows_per_w)])

        lane = lax.iota(jnp.int32, _B)
        lane_m3 = jnp.maximum(lane - 3, 0)

        def fire(b, s):
            off = b * _B
            xb = x_v[pl.ds(off, _B)]
            yb = y_v[pl.ds(off, _B)]
            zb = z_v[pl.ds(off, _B)]
            h = 248.0 * (yb / zb) + 112.0
            w = 248.0 * (xb / (-zb)) + 112.0
            h = jnp.clip(h, 0.0, 223.0)
            w = jnp.clip(w, 0.0, 223.0)
            wts = []
            for l in range(4):
                hh = _HS[l]
                x = h / _SCALES[l]
                y = w / _SCALES[l]
                xi1 = x.astype(jnp.int32)
                x1f = xi1.astype(jnp.float32)
                xfr = x > x1f
                xi2 = xi1 + jnp.where(xfr, 1, 0)
                x2f = x1f + jnp.where(xfr, 1.0, 0.0)
                yi1 = y.astype(jnp.int32)
                y1f = yi1.astype(jnp.float32)
                yfr = y > y1f
                yi2 = yi1 + jnp.where(yfr, 1, 0)
                y2f = y1f + jnp.where(yfr, 1.0, 0.0)
                xi1 = jnp.clip(xi1, 0, hh - 1)
                xi2 = jnp.clip(xi2, 0, hh - 1)
                yi1 = jnp.clip(yi1, 0, hh - 1)
                yi2 = jnp.clip(yi2, 0, hh - 1)
                wts.extend(((x2f - x) * (y2f - y), (x - x1f) * (y2f - y),
                            (x2f - x) * (y - y1f), (x - x1f) * (y - y1f)))
                iref = isets[s][l]
                iref[pl.ds(0, _B)] = xi1 * hh + yi1
                iref[pl.ds(_B, _B)] = xi2 * hh + yi1
                iref[pl.ds(2 * _B, _B)] = xi1 * hh + yi2
                iref[pl.ds(3 * _B, _B)] = xi2 * hh + yi2
            for l in range(4):
                pltpu.async_copy(tabs[l].at[isets[s][l]], qsets[s][l], sems[s])
            return wts + [xb, yb, zb]

        def compute(b, s, state):
            qs = qsets[s]
            xb, yb, zb = state[16], state[17], state[18]

            def point_body(i, carry2):
                for l in range(4):
                    q = qs[l]
                    w11 = _bcast(state[4 * l + 0], i)
                    w21 = _bcast(state[4 * l + 1], i)
                    w12 = _bcast(state[4 * l + 2], i)
                    w22 = _bcast(state[4 * l + 3], i)
                    if l < 2:
                        for k in range(_CS[l] // 16):
                            sl = pl.ds(k * 16, 16)
                            v = (q[i, sl] * w11 + q[i + _B, sl] * w21
                                 + q[i + 2 * _B, sl] * w12
                                 + q[i + 3 * _B, sl] * w22)
                            if l == 0 and k == 0:
                                head = jnp.where(
                                    lane == 0, _bcast(xb, i),
                                    jnp.where(lane == 1, _bcast(yb, i),
                                              jnp.where(lane == 2, _bcast(zb, i),
                                                        _permute(v, lane_m3))))
                                o_v[i, pl.ds(0, 16)] = head
                            o_v[i, pl.ds(_COL0[l] + k * 16, 16)] = v
                    else:
                        for k in range(_CS[l] // 32):
                            sl = pl.ds(k * 16, 16)
                            a11, b11 = _halves(q[i, sl])
                            a21, b21 = _halves(q[i + _B, sl])
                            a12, b12 = _halves(q[i + 2 * _B, sl])
                            a22, b22 = _halves(q[i + 3 * _B, sl])
                            va = a11 * w11 + a21 * w21 + a12 * w12 + a22 * w22
                            vb = b11 * w11 + b21 * w21 + b12 * w12 + b22 * w22
                            o_v[i, pl.ds(_COL0[l] + k * 32, 16)] = va
                            o_v[i, pl.ds(_COL0[l] + k * 32 + 16, 16)] = vb
                return carry2

            lax.fori_loop(0, _B, point_body, 0)

            @pl.when(b < nb)
            def _():
                pltpu.sync_copy(o_v, out_hbm.at[pl.ds(base + b * _B, _B)])

        def wait_set(s):
            for l in range(4):
                pltpu.make_async_copy(tabs[l].at[isets[s][l]], qsets[s][l],
                                      sems[s]).wait()

        state0 = fire(0, 0)

        def pair_body(g, carry):
            state = carry
            b0 = 2 * g
            state_n = fire(b0 + 1, 1)
            wait_set(0)
            compute(b0, 0, state)
            state_nn = fire(b0 + 2, 0)
            wait_set(1)
            compute(b0 + 1, 1, state_n)
            return state_nn

        lax.fori_loop(0, nb_max // 2, pair_body, state0)
        wait_set(0)

    return sc_call


def _prep_table(t, hh, cc, gw, dt):
    t = t.reshape(hh * hh, cc)
    if dt == jnp.float32:
        if gw != cc:
            t = jnp.pad(t, ((0, 0), (0, gw - cc)))
        return t
    tu = jax.lax.bitcast_convert_type(
        t.astype(jnp.bfloat16), jnp.uint16).astype(jnp.uint32)
    tu = tu.reshape(hh * hh, cc // 32, 2, 16)
    word = tu[:, :, 0, :] | (tu[:, :, 1, :] << 16)
    return jax.lax.bitcast_convert_type(word, jnp.int32).reshape(
        hh * hh, cc // 2)


@jax.jit
def kernel(inputs, img_feat_0, img_feat_1, img_feat_2, img_feat_3):
    n = inputs.shape[0]
    nw = 32
    rows_per_w = -(-n // (nw * _B)) * _B
    n_pad = nw * rows_per_w
    xyz = jnp.pad(inputs.T, ((0, 0), (0, n_pad - n)), constant_values=-1.0)
    tables = [
        _prep_table(t, hh, cc, gw, dt)
        for t, hh, cc, gw, dt in zip(
            (img_feat_0, img_feat_1, img_feat_2, img_feat_3),
            _HS, _CS, _GW, _QDT)
    ]
    return _make_sc_call(n)(xyz[0], xyz[1], xyz[2], *tables)
